# baseline (speedup 1.0000x reference)
import functools

import jax
import jax.numpy as jnp
from jax import lax
from jax.experimental import pallas as pl
from jax.experimental.pallas import tpu as pltpu
from jax.experimental.pallas import tpu_sc as plsc

ROI_D, ROI_H, ROI_W = 32, 64, 64
B, D, H, W, C = 2, 96, 256, 256, 2
N = 64
ROW = ROI_W * C
BLKS = 2
NUM_CROPS = B * N
NUM_WORKERS = 32
CROPS_PER_W = NUM_CROPS // NUM_WORKERS
ZCHUNK = 2
NCHUNK = ROI_D // ZCHUNK
VPR = ROW // 16
IN_ROWS = ZCHUNK * ROI_H * BLKS
ST_ROWS = ZCHUNK * ROI_H


def _roi_body(image_hbm, boxes_hbm, out_hbm,
              boxes_v, in0, in1, st0, st1, isem0, isem1, osem0, osem1):
    wid = lax.axis_index("s") * 2 + lax.axis_index("c")

    img = image_hbm

    pltpu.sync_copy(boxes_hbm, boxes_v)

    ins = (in0, in1)
    sts = (st0, st1)
    isems = (isem0, isem1)
    osems = (osem0, osem1)
    T = CROPS_PER_W * NCHUNK

    def crop_of(t):
        j = t // NCHUNK
        idx = wid * CROPS_PER_W + j
        b = idx // N
        n = idx % N
        v = boxes_v[idx, pl.ds(0, 16)]
        return b, n, v[0], v[1], v[2]

    def in_copy(t, p):
        b, n, z0, y0, x0 = crop_of(t)
        ch = t % NCHUNK
        return pltpu.make_async_copy(
            img.at[b, pl.ds(z0 + ch * ZCHUNK, ZCHUNK),
                   pl.ds(y0, ROI_H), pl.ds(0, BLKS), :],
            ins[p], isems[p])

    def in_wait(p):
        pltpu.make_async_copy(
            img.at[0, pl.ds(0, ZCHUNK), pl.ds(0, ROI_H), pl.ds(0, BLKS), :],
            ins[p], isems[p]).wait()

    def out_wait(p):
        pltpu.make_async_copy(
            sts[p], out_hbm.at[0, 0, pl.ds(0, ST_ROWS)], osems[p]).wait()

    def fix_and_out(t, p):
        b, n, _, _, x0 = crop_of(t)
        ch = t % NCHUNK
        q0 = x0 * C
        src = ins[p]
        dst = sts[p]
        cols = []
        for col in range(VPR):
            q = q0 + col * 16
            cols.append((q // ROW, q % ROW))

        @plsc.parallel_loop(0, ST_ROWS, unroll=2)
        def _(r):
            z = r // ROI_H
            y = r % ROI_H
            for col in range(VPR):
                s, qq = cols[col]
                dst[r, pl.ds(col * 16, 16)] = (
                    src[z, y, s, pl.ds(qq, 16)])

        pltpu.make_async_copy(
            dst, out_hbm.at[b, n, pl.ds(ch * ST_ROWS, ST_ROWS)],
            osems[p]).start()

    def pair(g, carry):
        for k in range(2):
            t = 2 * g + k

            @pl.when(t >= 2)
            def _(k=k):
                out_wait(k)

            in_copy(t, k).start()

            @pl.when(t >= 1)
            def _(t=t, k=k):
                in_wait(1 - k)
                fix_and_out(t - 1, 1 - k)
        return carry
    lax.fori_loop(0, T // 2, pair, 0)

    in_wait((T - 1) % 2)
    fix_and_out(T - 1, (T - 1) % 2)
    out_wait((T - 2) % 2)
    out_wait((T - 1) % 2)


def kernel(image, boxes):
    image_v = image.reshape(B, D, H, W * C // ROW, ROW)
    boxes_pad = jnp.pad(boxes.astype(jnp.int32).reshape(NUM_CROPS, 3),
                        ((0, 0), (0, 125)))
    run = functools.partial(
        pl.kernel,
        mesh=plsc.VectorSubcoreMesh(core_axis_name="c", subcore_axis_name="s"),
        compiler_params=pltpu.CompilerParams(use_tc_tiling_on_sc=False),
        out_type=jax.ShapeDtypeStruct((B, N, ROI_D * ROI_H, ROW),
                                      jnp.float32),
        scratch_types=[
            pltpu.VMEM((NUM_CROPS, 128), jnp.int32),
            pltpu.VMEM((ZCHUNK, ROI_H, BLKS, ROW), jnp.float32),
            pltpu.VMEM((ZCHUNK, ROI_H, BLKS, ROW), jnp.float32),
            pltpu.VMEM((ST_ROWS, ROW), jnp.float32),
            pltpu.VMEM((ST_ROWS, ROW), jnp.float32),
            pltpu.SemaphoreType.DMA,
            pltpu.SemaphoreType.DMA,
            pltpu.SemaphoreType.DMA,
            pltpu.SemaphoreType.DMA,
        ],
    )(_roi_body)
    out4 = run(image_v, boxes_pad)
    return out4.reshape(B, N, ROI_D, ROI_H, ROI_W, C)
